# baseline (speedup 1.0000x reference)
import functools

import jax
import jax.numpy as jnp
from jax import lax
from jax.experimental import pallas as pl
from jax.experimental.pallas import tpu as pltpu
from jax.experimental.pallas import tpu_sc as plsc

VOCAB = 1000000
D = 64
BATCH = 4096
HIST = 50
NC = 2
NS = 16
NW = NC * NS
CHUNK = BATCH // NW
L = 16
NBUF = 4
N_GROUPS = HIST // NBUF
N_TAIL = HIST - N_GROUPS * NBUF


def _emb_body(idx_hbm, table_hbm, out_hbm, idx_v, idx2_v, rows_v, packed_v,
              *sems):
    gsems, osems = sems[:NBUF], sems[NBUF:]
    wid = lax.axis_index("s") * NC + lax.axis_index("c")
    b0 = wid * CHUNK
    pltpu.sync_copy(idx_hbm.at[:, pl.ds(b0, CHUNK)], idx_v)

    def shift_row(r, carry):
        for j in range(CHUNK // L):
            idx2_v[r, pl.ds(j * L, L)] = idx_v[r, pl.ds(j * L, L)] >> 1
        return carry

    lax.fori_loop(0, HIST, shift_row, 0)

    def g_desc(k, b):
        return pltpu.make_async_copy(
            table_hbm.at[idx2_v.at[k]], rows_v.at[b], gsems[b])

    def o_desc(k, b):
        return pltpu.make_async_copy(
            packed_v.at[b], out_hbm.at[k, :, pl.ds(b0, CHUNK)], osems[b])

    iota = lax.iota(jnp.int32, L)

    def transform(k, b):
        def rgroup(rg, carry):
            raw = idx_v[k, pl.ds(rg * L, L)]
            cols0 = (raw & 1) * D
            rows16 = iota + rg * L
            for c in range(D):
                vals = plsc.load_gather(rows_v.at[b], [rows16, cols0 + c])
                packed_v.at[b][c, pl.ds(rg * L, L)] = vals
            return carry

        lax.fori_loop(0, CHUNK // L, rgroup, 0)

    for b in range(NBUF):
        g_desc(b, b).start()

    def group(g, carry):
        for b in range(NBUF):
            k = g * NBUF + b
            g_desc(k, b).wait()
            @pl.when(g > 0)
            def _():
                o_desc(k, b).wait()
            transform(k, b)
            o_desc(k, b).start()
            @pl.when(k + NBUF < HIST)
            def _():
                g_desc(k + NBUF, b).start()
        return carry

    lax.fori_loop(0, N_GROUPS, group, 0)

    for b in range(N_TAIL):
        k = N_GROUPS * NBUF + b
        g_desc(k, b).wait()
        o_desc(k, b).wait()
        transform(k, b)
        o_desc(k, b).start()

    for b in range(NBUF):
        o_desc(0, b).wait()


@jax.jit
def kernel(x, table):
    xt = x.T.astype(jnp.int32)
    t2 = table.reshape(VOCAB // 2, 2 * D)
    run = pl.kernel(
        _emb_body,
        mesh=plsc.VectorSubcoreMesh(core_axis_name="c", subcore_axis_name="s"),
        out_type=jax.ShapeDtypeStruct((HIST, D, BATCH), jnp.float32),
        scratch_types=[
            pltpu.VMEM((HIST, CHUNK), jnp.int32),
            pltpu.VMEM((HIST, CHUNK), jnp.int32),
            pltpu.VMEM((NBUF, CHUNK, 2 * D), jnp.float32),
            pltpu.VMEM((NBUF, D, CHUNK), jnp.float32),
        ] + [pltpu.SemaphoreType.DMA] * (2 * NBUF),
        compiler_params=pltpu.CompilerParams(
            use_tc_tiling_on_sc=True, needs_layout_passes=False),
    )
    out_t = run(xt, t2)
    return jnp.transpose(out_t, (2, 0, 1))


# speedup vs baseline: 1.1289x; 1.1289x over previous
import functools

import jax
import jax.numpy as jnp
from jax import lax
from jax.experimental import pallas as pl
from jax.experimental.pallas import tpu as pltpu
from jax.experimental.pallas import tpu_sc as plsc

VOCAB = 1000000
D = 64
BATCH = 4096
HIST = 50
NC = 2
NS = 16
NW = NC * NS
CHUNK = BATCH // NW
L = 16
NBUF = 4
N_GROUPS = HIST // NBUF
N_TAIL = HIST - N_GROUPS * NBUF


def _emb_body(idx_hbm, table_hbm, out_hbm, idx_v, idx2_v, rows_v, packed_v,
              *sems):
    gsems, osems = sems[:NBUF], sems[NBUF:]
    wid = lax.axis_index("s") * NC + lax.axis_index("c")
    b0 = wid * CHUNK
    pltpu.sync_copy(idx_hbm.at[:, pl.ds(b0, CHUNK)], idx_v)

    def shift_row(r, carry):
        for j in range(CHUNK // L):
            idx2_v[r, pl.ds(j * L, L)] = idx_v[r, pl.ds(j * L, L)] >> 1
        return carry

    lax.fori_loop(0, HIST, shift_row, 0)

    def g_desc(k, b):
        return pltpu.make_async_copy(
            table_hbm.at[idx2_v.at[k]], rows_v.at[b], gsems[b])

    def o_desc(k, b):
        return pltpu.make_async_copy(
            packed_v.at[b], out_hbm.at[k, :, pl.ds(b0, CHUNK)], osems[b])

    iota = lax.iota(jnp.int32, L)

    def transform(k, b):
        @plsc.parallel_loop(0, CHUNK // L, unroll=2)
        def _rgroup(rg):
            raw = idx_v[k, pl.ds(rg * L, L)]
            cols0 = (raw & 1) * D
            rows16 = iota + rg * L
            for c in range(D):
                vals = plsc.load_gather(rows_v.at[b], [rows16, cols0 + c])
                packed_v.at[b][c, pl.ds(rg * L, L)] = vals

    for b in range(NBUF):
        g_desc(b, b).start()

    def group(g, carry):
        for b in range(NBUF):
            k = g * NBUF + b
            g_desc(k, b).wait()
            @pl.when(g > 0)
            def _():
                o_desc(k, b).wait()
            transform(k, b)
            o_desc(k, b).start()
            @pl.when(k + NBUF < HIST)
            def _():
                g_desc(k + NBUF, b).start()
        return carry

    lax.fori_loop(0, N_GROUPS, group, 0)

    for b in range(N_TAIL):
        k = N_GROUPS * NBUF + b
        g_desc(k, b).wait()
        o_desc(k, b).wait()
        transform(k, b)
        o_desc(k, b).start()

    for b in range(NBUF):
        o_desc(0, b).wait()


@jax.jit
def kernel(x, table):
    xt = x.T.astype(jnp.int32)
    t2 = table.reshape(VOCAB // 2, 2 * D)
    run = pl.kernel(
        _emb_body,
        mesh=plsc.VectorSubcoreMesh(core_axis_name="c", subcore_axis_name="s"),
        out_type=jax.ShapeDtypeStruct((HIST, D, BATCH), jnp.float32),
        scratch_types=[
            pltpu.VMEM((HIST, CHUNK), jnp.int32),
            pltpu.VMEM((HIST, CHUNK), jnp.int32),
            pltpu.VMEM((NBUF, CHUNK, 2 * D), jnp.float32),
            pltpu.VMEM((NBUF, D, CHUNK), jnp.float32),
        ] + [pltpu.SemaphoreType.DMA] * (2 * NBUF),
        compiler_params=pltpu.CompilerParams(
            use_tc_tiling_on_sc=True, needs_layout_passes=False),
    )
    out_t = run(xt, t2)
    return jnp.transpose(out_t, (2, 0, 1))


# speedup vs baseline: 1.2930x; 1.1454x over previous
import functools

import jax
import jax.numpy as jnp
from jax import lax
from jax.experimental import pallas as pl
from jax.experimental.pallas import tpu as pltpu
from jax.experimental.pallas import tpu_sc as plsc

VOCAB = 1000000
D = 64
BATCH = 4096
HIST = 50
NC = 2
NS = 16
NW = NC * NS
CHUNK = BATCH // NW
L = 16
NBUF = 5
N_GROUPS = HIST // NBUF


def _emb_body(idx_hbm, table_hbm, out_hbm, idx_v, rows_v, *sems):
    gsems, osems = sems[:NBUF], sems[NBUF:]
    wid = lax.axis_index("s") * NC + lax.axis_index("c")
    b0 = wid * CHUNK
    pltpu.sync_copy(idx_hbm.at[:, pl.ds(b0, CHUNK)], idx_v)

    @plsc.parallel_loop(0, HIST)
    def _double_row(r):
        for j in range(CHUNK // L):
            idx_v[r, pl.ds(j * L, L)] = idx_v[r, pl.ds(j * L, L)] * 2

    def g_desc(k, b):
        return pltpu.make_async_copy(
            table_hbm.at[idx_v.at[k]], rows_v.at[b], gsems[b])

    def o_desc(k, b):
        return pltpu.make_async_copy(
            rows_v.at[b], out_hbm.at[pl.ds(b0, CHUNK), k], osems[b])

    for b in range(NBUF):
        g_desc(b, b).start()

    def group(g, carry):
        for b in range(NBUF):
            k = g * NBUF + b
            g_desc(k, b).wait()
            o_desc(k, b).start()
            o_desc(k, b).wait()
            g_desc(k + NBUF, b).start()
        return carry

    lax.fori_loop(0, N_GROUPS - 1, group, 0)

    for b in range(NBUF):
        k = (N_GROUPS - 1) * NBUF + b
        g_desc(k, b).wait()
        o_desc(k, b).start()
    for b in range(NBUF):
        k = (N_GROUPS - 1) * NBUF + b
        o_desc(k, b).wait()


@jax.jit
def kernel(x, table):
    xt = x.T.astype(jnp.int32)
    tp = jnp.pad(table, ((0, 0), (0, D))).reshape(2 * VOCAB, D)
    run = pl.kernel(
        _emb_body,
        mesh=plsc.VectorSubcoreMesh(core_axis_name="c", subcore_axis_name="s"),
        out_type=jax.ShapeDtypeStruct((BATCH, HIST, D), jnp.float32),
        scratch_types=[
            pltpu.VMEM((HIST, CHUNK), jnp.int32),
            pltpu.VMEM((NBUF, CHUNK, D), jnp.float32),
        ] + [pltpu.SemaphoreType.DMA] * (2 * NBUF),
        compiler_params=pltpu.CompilerParams(use_tc_tiling_on_sc=False),
    )
    return run(xt, tp)
